# baseline (speedup 1.0000x reference)
import functools

import jax
import jax.numpy as jnp
from jax import lax
from jax.experimental import pallas as pl
from jax.experimental.pallas import tpu as pltpu
from jax.experimental.pallas import tpu_sc as plsc

M = 2048
N = 16384
K = 16
NC, NS, L = 2, 16, 16
NW = NC * NS
QPW = N // NW
CHUNKS = QPW // L
U = 3
SB = 2048


def _rank_body(krow_ref, out_ref):
    i0 = pl.program_id(0)
    krow = krow_ref[...]
    kcol = jnp.transpose(krow_ref[:, pl.ds(i0 * SB, SB)], (1, 0))
    lt = krow < kcol
    eq = krow == kcol
    j = lax.broadcasted_iota(jnp.int32, (SB, M), 1)
    i = lax.broadcasted_iota(jnp.int32, (SB, M), 0) + i0 * SB
    before = jnp.logical_or(lt, jnp.logical_and(eq, j < i))
    bf = jnp.where(before, 1.0, 0.0).astype(jnp.float32)
    rank = jax.lax.dot_general(
        bf, jnp.ones((M, 1), jnp.float32),
        (((1,), (0,)), ((), ())),
        preferred_element_type=jnp.float32,
    )
    out_ref[...] = jnp.transpose(rank, (1, 0)).astype(jnp.int32)


def _rank_keys(xm):
    out = pl.pallas_call(
        _rank_body,
        grid=(M // SB,),
        in_specs=[pl.BlockSpec((1, M), lambda i: (0, 0))],
        out_specs=pl.BlockSpec((1, SB), lambda i: (0, i)),
        out_shape=jax.ShapeDtypeStruct((1, M), jnp.int32),
    )(xm.reshape(1, M))
    return out.reshape(M)


def _knn_body(
    x_hbm, keys_hbm, ranks_hbm, out_hbm,
    keys_v, raw_v, rk_v, xq_v, out_v, sem_k, sem_r, sem_x,
):
    wid = lax.axis_index("s") * NC + lax.axis_index("c")
    base = wid * QPW
    cp_k = pltpu.make_async_copy(keys_hbm, raw_v, sem_k)
    cp_r = pltpu.make_async_copy(ranks_hbm, rk_v, sem_r)
    cp_x = pltpu.make_async_copy(x_hbm.at[pl.ds(base, QPW)], xq_v, sem_x)
    cp_k.start()
    cp_r.start()
    cp_x.start()
    keys_v[pl.ds(0, L)] = jnp.full((L,), -jnp.inf, jnp.float32)
    keys_v[pl.ds(M + K, L)] = jnp.full((L,), jnp.inf, jnp.float32)
    cp_k.wait()
    cp_r.wait()
    cp_x.wait()
    lanes = lax.iota(jnp.int32, L)
    half = M // 2

    @plsc.parallel_loop(0, M // L, unroll=4)
    def _place(pi):
        o = pi * L
        plsc.store_scatter(
            keys_v, [rk_v[pl.ds(o, L)] + K], raw_v[pl.ds(o, L)]
        )

    @plsc.parallel_loop(0, CHUNKS, unroll=U)
    def _chunk(ci):
        off = ci * L
        xq = xq_v[pl.ds(off, L)]
        rows = off + lanes
        kv = plsc.load_gather(keys_v, [jnp.full((L,), K + half - 1, jnp.int32)])
        posK = jnp.where(kv < xq, K + half, K)
        b = half // 2
        while b >= 1:
            kv = plsc.load_gather(keys_v, [posK + (b - 1)])
            posK = jnp.where(kv < xq, posK + b, posK)
            b //= 2
        kv = plsc.load_gather(keys_v, [posK])
        posK = jnp.where(kv < xq, posK + 1, posK)
        plsc.store_scatter(out_v, [rows, jnp.zeros((L,), jnp.int32)], xq)
        l = posK - 1
        r = posK
        dl = jnp.abs(xq - plsc.load_gather(keys_v, [l]))
        dr = jnp.abs(plsc.load_gather(keys_v, [r]) - xq)
        for t in range(K):
            tk = dl < dr
            d = jnp.where(tk, dl, dr)
            plsc.store_scatter(out_v, [rows, jnp.full((L,), t + 1, jnp.int32)], d)
            if t < K - 1:
                l = jnp.where(tk, l - 1, l)
                r = jnp.where(tk, r, r + 1)
                nd = jnp.abs(xq - plsc.load_gather(keys_v, [jnp.where(tk, l, r)]))
                dl = jnp.where(tk, nd, dl)
                dr = jnp.where(tk, dr, nd)

    pltpu.sync_copy(out_v, out_hbm.at[pl.ds(base, QPW)])


@functools.cache
def _make_knn():
    mesh = plsc.VectorSubcoreMesh(
        core_axis_name="c", subcore_axis_name="s", num_cores=NC, num_subcores=NS
    )
    return pl.kernel(
        _knn_body,
        out_type=jax.ShapeDtypeStruct((N, K + 1), jnp.float32),
        mesh=mesh,
        scratch_types=[
            pltpu.VMEM((M + 2 * K,), jnp.float32),
            pltpu.VMEM((M,), jnp.float32),
            pltpu.VMEM((M,), jnp.int32),
            pltpu.VMEM((QPW,), jnp.float32),
            pltpu.VMEM((QPW, K + 1), jnp.float32),
            pltpu.SemaphoreType.DMA,
            pltpu.SemaphoreType.DMA,
            pltpu.SemaphoreType.DMA,
        ],
        compiler_params=pltpu.CompilerParams(needs_layout_passes=False),
    )


def kernel(x, x_measured):
    ranks = _rank_keys(x_measured)
    return _make_knn()(x, x_measured, ranks)


# speedup vs baseline: 1.0293x; 1.0293x over previous
import functools

import jax
import jax.numpy as jnp
from jax import lax
from jax.experimental import pallas as pl
from jax.experimental.pallas import tpu as pltpu
from jax.experimental.pallas import tpu_sc as plsc

M = 2048
N = 16384
K = 16
NC, NS, L = 2, 16, 16
NW = NC * NS
QPW = N // NW
CHUNKS = QPW // L
U = 2
SB = 2048


def _rank_body(krow_ref, out_ref):
    i0 = pl.program_id(0)
    krow = krow_ref[...]
    kcol = jnp.transpose(krow_ref[:, pl.ds(i0 * SB, SB)], (1, 0))
    lt = krow < kcol
    eq = krow == kcol
    j = lax.broadcasted_iota(jnp.int32, (SB, M), 1)
    i = lax.broadcasted_iota(jnp.int32, (SB, M), 0) + i0 * SB
    before = jnp.logical_or(lt, jnp.logical_and(eq, j < i))
    bf = jnp.where(before, 1.0, 0.0).astype(jnp.float32)
    rank = jax.lax.dot_general(
        bf, jnp.ones((M, 1), jnp.float32),
        (((1,), (0,)), ((), ())),
        preferred_element_type=jnp.float32,
    )
    out_ref[...] = jnp.transpose(rank, (1, 0)).astype(jnp.int32)


def _rank_keys(xm):
    out = pl.pallas_call(
        _rank_body,
        grid=(M // SB,),
        in_specs=[pl.BlockSpec((1, M), lambda i: (0, 0))],
        out_specs=pl.BlockSpec((1, SB), lambda i: (0, i)),
        out_shape=jax.ShapeDtypeStruct((1, M), jnp.int32),
    )(xm.reshape(1, M))
    return out.reshape(M)


def _knn_body(
    x_hbm, keys_hbm, ranks_hbm, out_hbm,
    keys_v, raw_v, rk_v, xq_v, out_v, sem_k, sem_r, sem_x,
):
    wid = lax.axis_index("s") * NC + lax.axis_index("c")
    base = wid * QPW
    cp_k = pltpu.make_async_copy(keys_hbm, raw_v, sem_k)
    cp_r = pltpu.make_async_copy(ranks_hbm, rk_v, sem_r)
    cp_x = pltpu.make_async_copy(x_hbm.at[pl.ds(base, QPW)], xq_v, sem_x)
    cp_k.start()
    cp_r.start()
    cp_x.start()
    keys_v[pl.ds(0, L)] = jnp.full((L,), -jnp.inf, jnp.float32)
    keys_v[pl.ds(M + K, L)] = jnp.full((L,), jnp.inf, jnp.float32)
    cp_k.wait()
    cp_r.wait()
    cp_x.wait()
    lanes = lax.iota(jnp.int32, L)
    half = M // 2

    @plsc.parallel_loop(0, M // L, unroll=4)
    def _place(pi):
        o = pi * L
        plsc.store_scatter(
            keys_v, [rk_v[pl.ds(o, L)] + K], raw_v[pl.ds(o, L)]
        )

    @plsc.parallel_loop(0, CHUNKS, unroll=U)
    def _chunk(ci):
        off = ci * L
        xq = xq_v[pl.ds(off, L)]
        rows = off + lanes
        kv = plsc.load_gather(keys_v, [jnp.full((L,), K + half - 1, jnp.int32)])
        posK = jnp.where(kv < xq, K + half, K)
        b = half // 2
        while b >= 1:
            kv = plsc.load_gather(keys_v, [posK + (b - 1)])
            posK = jnp.where(kv < xq, posK + b, posK)
            b //= 2
        kv = plsc.load_gather(keys_v, [posK])
        posK = jnp.where(kv < xq, posK + 1, posK)
        plsc.store_scatter(out_v, [rows, jnp.zeros((L,), jnp.int32)], xq)
        l = posK - 1
        r = posK
        dl = jnp.abs(xq - plsc.load_gather(keys_v, [l]))
        dr = jnp.abs(plsc.load_gather(keys_v, [r]) - xq)
        for t in range(K):
            tk = dl < dr
            d = jnp.where(tk, dl, dr)
            plsc.store_scatter(out_v, [rows, jnp.full((L,), t + 1, jnp.int32)], d)
            if t < K - 1:
                l = jnp.where(tk, l - 1, l)
                r = jnp.where(tk, r, r + 1)
                nd = jnp.abs(xq - plsc.load_gather(keys_v, [jnp.where(tk, l, r)]))
                dl = jnp.where(tk, nd, dl)
                dr = jnp.where(tk, dr, nd)

    pltpu.sync_copy(out_v, out_hbm.at[pl.ds(base, QPW)])


@functools.cache
def _make_knn():
    mesh = plsc.VectorSubcoreMesh(
        core_axis_name="c", subcore_axis_name="s", num_cores=NC, num_subcores=NS
    )
    return pl.kernel(
        _knn_body,
        out_type=jax.ShapeDtypeStruct((N, K + 1), jnp.float32),
        mesh=mesh,
        scratch_types=[
            pltpu.VMEM((M + 2 * K,), jnp.float32),
            pltpu.VMEM((M,), jnp.float32),
            pltpu.VMEM((M,), jnp.int32),
            pltpu.VMEM((QPW,), jnp.float32),
            pltpu.VMEM((QPW, K + 1), jnp.float32),
            pltpu.SemaphoreType.DMA,
            pltpu.SemaphoreType.DMA,
            pltpu.SemaphoreType.DMA,
        ],
        compiler_params=pltpu.CompilerParams(needs_layout_passes=False),
    )


def kernel(x, x_measured):
    ranks = _rank_keys(x_measured)
    return _make_knn()(x, x_measured, ranks)
